# baseline (speedup 1.0000x reference)
import functools

import jax
import jax.numpy as jnp
from jax import lax
from jax.experimental import pallas as pl
from jax.experimental.pallas import tpu as pltpu
from jax.experimental.pallas import tpu_sc as plsc

_CHUNK = 64


def _sc_gather(idx_flat, token_table):
    N = idx_flat.shape[0]
    V, H = token_table.shape

    info = plsc.get_sparse_core_info()
    nw = info.num_cores * info.num_subcores
    rpw = N // nw
    nch = rpw // _CHUNK

    mesh = plsc.VectorSubcoreMesh(core_axis_name="c", subcore_axis_name="s")

    @functools.partial(
        pl.kernel,
        out_type=jax.ShapeDtypeStruct((N, H), jnp.float32),
        mesh=mesh,
        compiler_params=pltpu.CompilerParams(needs_layout_passes=False),
        scratch_types=[
            pltpu.VMEM((rpw,), jnp.int32),
            pltpu.VMEM((_CHUNK, H), jnp.float32),
            pltpu.VMEM((_CHUNK, H), jnp.float32),
            pltpu.SemaphoreType.DMA,
            pltpu.SemaphoreType.DMA,
        ],
    )
    def sc_kernel(idx_hbm, table_hbm, out_hbm, idx_v, buf0, buf1, sem0, sem1):
        wid = lax.axis_index("s") * info.num_cores + lax.axis_index("c")
        base = wid * rpw
        pltpu.sync_copy(idx_hbm.at[pl.ds(base, rpw)], idx_v)
        bufs = (buf0, buf1)
        sems = (sem0, sem1)
        cps = [None] * nch
        cps[0] = pltpu.async_copy(
            table_hbm.at[idx_v.at[pl.ds(0, _CHUNK)]], bufs[0], sems[0])
        for c in range(nch):
            if c + 1 < nch:
                cps[c + 1] = pltpu.async_copy(
                    table_hbm.at[idx_v.at[pl.ds((c + 1) * _CHUNK, _CHUNK)]],
                    bufs[(c + 1) % 2], sems[(c + 1) % 2])
            cps[c].wait()
            pltpu.sync_copy(bufs[c % 2],
                            out_hbm.at[pl.ds(base + c * _CHUNK, _CHUNK)])

    return sc_kernel(idx_flat, token_table)


def _tc_ln_body(tok_ref, pos_ref, g_ref, b_ref, *rest):
    out_ref = rest[-1]
    x = tok_ref[0] + pos_ref[...]
    mean = jnp.mean(x, axis=-1, keepdims=True)
    xc = x - mean
    var = jnp.mean(xc * xc, axis=-1, keepdims=True)
    y = xc * lax.rsqrt(var + 1e-5)
    out_ref[0] = y * g_ref[...] + b_ref[...]


_NCH = 4
_R = 512


def kernel(input_token, token_table, pos_table, ln_gamma, ln_beta):
    B, L = input_token.shape
    V, H = token_table.shape
    lc = L // _NCH

    toks = []
    for k in range(_NCH):
        idx_k = input_token[:, k * lc:(k + 1) * lc].reshape(-1)
        toks.append(_sc_gather(idx_k, token_table).reshape(B, lc, H))

    g2 = ln_gamma.reshape(1, H)
    b2 = ln_beta.reshape(1, H)
    nb = lc // _R
    out = None
    for k in range(_NCH):
        off = k * nb
        in_specs = [
            pl.BlockSpec((1, _R, H), lambda i, b: (b, i, 0)),
            pl.BlockSpec((_R, H), lambda i, b, o=off: (i + o, 0)),
            pl.BlockSpec((1, H), lambda i, b: (0, 0)),
            pl.BlockSpec((1, H), lambda i, b: (0, 0)),
        ]
        args = [toks[k], pos_table, g2, b2]
        aliases = {}
        if out is not None:
            in_specs.append(pl.BlockSpec((1, 8, 128), lambda i, b: (0, 0, 0)))
            args.append(out)
            aliases = {4: 0}
        out = pl.pallas_call(
            _tc_ln_body,
            grid=(nb, B),
            in_specs=in_specs,
            out_specs=pl.BlockSpec(
                (1, _R, H), lambda i, b, o=off: (b, i + o, 0)),
            out_shape=jax.ShapeDtypeStruct((B, L, H), jnp.float32),
            input_output_aliases=aliases,
        )(*args)
    return out


# speedup vs baseline: 1.0804x; 1.0804x over previous
import functools

import jax
import jax.numpy as jnp
from jax import lax
from jax.experimental import pallas as pl
from jax.experimental.pallas import tpu as pltpu
from jax.experimental.pallas import tpu_sc as plsc

_CHUNK = 64


def _sc_gather(idx_flat, token_table):
    N = idx_flat.shape[0]
    V, H = token_table.shape

    info = plsc.get_sparse_core_info()
    nw = info.num_cores * info.num_subcores
    rpw = N // nw
    nch = rpw // _CHUNK

    mesh = plsc.VectorSubcoreMesh(core_axis_name="c", subcore_axis_name="s")

    @functools.partial(
        pl.kernel,
        out_type=jax.ShapeDtypeStruct((N, H), jnp.float32),
        mesh=mesh,
        compiler_params=pltpu.CompilerParams(needs_layout_passes=False),
        scratch_types=[
            pltpu.VMEM((rpw,), jnp.int32),
            pltpu.VMEM((_CHUNK, H), jnp.float32),
            pltpu.VMEM((_CHUNK, H), jnp.float32),
            pltpu.SemaphoreType.DMA,
            pltpu.SemaphoreType.DMA,
        ],
    )
    def sc_kernel(idx_hbm, table_hbm, out_hbm, idx_v, buf0, buf1, sem0, sem1):
        wid = lax.axis_index("s") * info.num_cores + lax.axis_index("c")
        base = wid * rpw
        pltpu.sync_copy(idx_hbm.at[pl.ds(base, rpw)], idx_v)
        bufs = (buf0, buf1)
        sems = (sem0, sem1)
        cps = [None] * nch
        cps[0] = pltpu.async_copy(
            table_hbm.at[idx_v.at[pl.ds(0, _CHUNK)]], bufs[0], sems[0])
        for c in range(nch):
            if c + 1 < nch:
                cps[c + 1] = pltpu.async_copy(
                    table_hbm.at[idx_v.at[pl.ds((c + 1) * _CHUNK, _CHUNK)]],
                    bufs[(c + 1) % 2], sems[(c + 1) % 2])
            cps[c].wait()
            pltpu.sync_copy(bufs[c % 2],
                            out_hbm.at[pl.ds(base + c * _CHUNK, _CHUNK)])

    return sc_kernel(idx_flat, token_table)


def _tc_ln_body(tok_ref, pos_ref, g_ref, b_ref, *rest):
    out_ref = rest[-1]
    x = tok_ref[0] + pos_ref[...]
    mean = jnp.mean(x, axis=-1, keepdims=True)
    xc = x - mean
    var = jnp.mean(xc * xc, axis=-1, keepdims=True)
    y = xc * lax.rsqrt(var + 1e-5)
    out_ref[0] = y * g_ref[...] + b_ref[...]


_NCH = 2
_R = 512


def kernel(input_token, token_table, pos_table, ln_gamma, ln_beta):
    B, L = input_token.shape
    V, H = token_table.shape
    lc = L // _NCH

    toks = []
    for k in range(_NCH):
        idx_k = input_token[:, k * lc:(k + 1) * lc].reshape(-1)
        toks.append(_sc_gather(idx_k, token_table).reshape(B, lc, H))

    g2 = ln_gamma.reshape(1, H)
    b2 = ln_beta.reshape(1, H)
    nb = lc // _R
    out = None
    for k in range(_NCH):
        off = k * nb
        in_specs = [
            pl.BlockSpec((1, _R, H), lambda i, b: (b, i, 0)),
            pl.BlockSpec((_R, H), lambda i, b, o=off: (i + o, 0)),
            pl.BlockSpec((1, H), lambda i, b: (0, 0)),
            pl.BlockSpec((1, H), lambda i, b: (0, 0)),
        ]
        args = [toks[k], pos_table, g2, b2]
        aliases = {}
        if out is not None:
            in_specs.append(pl.BlockSpec((1, 8, 128), lambda i, b: (0, 0, 0)))
            args.append(out)
            aliases = {4: 0}
        out = pl.pallas_call(
            _tc_ln_body,
            grid=(nb, B),
            in_specs=in_specs,
            out_specs=pl.BlockSpec(
                (1, _R, H), lambda i, b, o=off: (b, i + o, 0)),
            out_shape=jax.ShapeDtypeStruct((B, L, H), jnp.float32),
            input_output_aliases=aliases,
        )(*args)
    return out
